# Initial kernel scaffold; baseline (speedup 1.0000x reference)
import jax
import jax.numpy as jnp
from jax.experimental import pallas as pl


def kernel(text_features, audio_features, video_features, W_text, b_text, W_audio, b_audio, W_video, b_video, Wl1, bl1, Wr1, br1, att1, bias1, Wl2, bl2, Wr2, br2, att2, bias2):
    raise NotImplementedError("write your pallas kernel here")


# speedup vs baseline: 223.9222x; 223.9222x over previous
import functools

import jax
import jax.numpy as jnp
from jax import lax
from jax.experimental import pallas as pl

B = 16384
UNI = 128
HID = 64
HEADS = 4
C1 = HID // HEADS
BT = 1024


def _leaky(x):
    return jnp.where(x >= 0, x, 0.2 * x)


def _elu(x):
    return jnp.where(x > 0, x, jnp.exp(x) - 1.0)


def _gnn_kernel(t_ref, a_ref, v_ref,
                wt_ref, bt_ref, wa_ref, ba_ref, wv_ref, bv_ref,
                wl1_ref, bl1_ref, wr1_ref, br1_ref, att1_ref, bias1_ref,
                wl2_ref, bl2_ref, wr2_ref, br2_ref, att2_ref,
                bias2_ref, out_ref):
    f32 = jnp.float32

    r = lax.broadcasted_iota(jnp.int32, (HID, HID), 0) // C1
    c = lax.broadcasted_iota(jnp.int32, (HID, HID), 1) // C1
    M = (r == c).astype(f32)

    x0 = jnp.dot(t_ref[...], wt_ref[...], preferred_element_type=f32) + bt_ref[...]
    x1 = jnp.dot(a_ref[...], wa_ref[...], preferred_element_type=f32) + ba_ref[...]
    x2 = jnp.dot(v_ref[...], wv_ref[...], preferred_element_type=f32) + bv_ref[...]
    x = (x0, x1, x2)

    wl1 = wl1_ref[...]
    wr1 = wr1_ref[...]
    bl1 = bl1_ref[...]
    br1 = br1_ref[...]
    att1 = att1_ref[...]
    xl = [jnp.dot(x[i], wl1, preferred_element_type=f32) + bl1 for i in range(3)]
    xr = [jnp.dot(x[i], wr1, preferred_element_type=f32) + br1 for i in range(3)]

    def logit1(s, d):
        e = _leaky(xl[s] + xr[d]) * att1
        return jnp.dot(e, M, preferred_element_type=f32)

    h = []
    bias1 = bias1_ref[...]
    for d in range(3):
        s1, s2 = [s for s in range(3) if s != d]
        l1 = logit1(s1, d)
        l2 = logit1(s2, d)
        m = jnp.maximum(l1, l2)
        e1 = jnp.exp(l1 - m)
        e2 = jnp.exp(l2 - m)
        den = e1 + e2 + 1e-16
        agg = (e1 * xl[s1] + e2 * xl[s2]) / den
        h.append(_elu(agg + bias1))

    wl2 = wl2_ref[...]
    wr2 = wr2_ref[...]
    bl2 = bl2_ref[...]
    br2 = br2_ref[...]
    att2 = att2_ref[...]
    bias2 = bias2_ref[...]
    yl = [jnp.dot(h[i], wl2, preferred_element_type=f32) + bl2 for i in range(3)]
    yr = [jnp.dot(h[i], wr2, preferred_element_type=f32) + br2 for i in range(3)]

    def logit2(s, d):
        e = _leaky(yl[s] + yr[d]) * att2
        return jnp.sum(e, axis=-1, keepdims=True)

    acc = jnp.zeros_like(yl[0])
    for d in range(3):
        s1, s2 = [s for s in range(3) if s != d]
        l1 = logit2(s1, d)
        l2 = logit2(s2, d)
        m = jnp.maximum(l1, l2)
        e1 = jnp.exp(l1 - m)
        e2 = jnp.exp(l2 - m)
        den = e1 + e2 + 1e-16
        acc = acc + (e1 * yl[s1] + e2 * yl[s2]) / den + bias2

    out_ref[...] = acc * (1.0 / 3.0)


@jax.jit
def kernel(text_features, audio_features, video_features, W_text, b_text,
           W_audio, b_audio, W_video, b_video, Wl1, bl1, Wr1, br1, att1,
           bias1, Wl2, bl2, Wr2, br2, att2, bias2):
    f32 = jnp.float32
    row = lambda b: b.reshape(1, -1).astype(f32)
    att1_flat = att1.reshape(1, HEADS * C1).astype(f32)

    grid = (B // BT,)
    data_spec = pl.BlockSpec((BT, UNI), lambda i: (i, 0))
    w_uni = pl.BlockSpec((UNI, HID), lambda i: (0, 0))
    w_hid = pl.BlockSpec((HID, HID), lambda i: (0, 0))
    vec = pl.BlockSpec((1, HID), lambda i: (0, 0))

    out = pl.pallas_call(
        _gnn_kernel,
        grid=grid,
        in_specs=[
            data_spec, data_spec, data_spec,
            w_uni, vec, w_uni, vec, w_uni, vec,
            w_hid, vec, w_hid, vec, vec, vec,
            w_hid, vec, w_hid, vec, vec,
            vec,
        ],
        out_specs=pl.BlockSpec((BT, HID), lambda i: (i, 0)),
        out_shape=jax.ShapeDtypeStruct((B, HID), f32),
    )(
        text_features, audio_features, video_features,
        W_text.T.astype(f32), row(b_text),
        W_audio.T.astype(f32), row(b_audio),
        W_video.T.astype(f32), row(b_video),
        Wl1.T.astype(f32), row(bl1),
        Wr1.T.astype(f32), row(br1), att1_flat, row(bias1),
        Wl2.T.astype(f32), row(bl2),
        Wr2.T.astype(f32), row(br2), att2.reshape(1, HID).astype(f32),
        row(bias2),
    )
    return out


# speedup vs baseline: 243.4058x; 1.0870x over previous
import jax
import jax.numpy as jnp
from jax.experimental import pallas as pl

B = 16384
UNI = 128
HID = 64
HEADS = 4
C1 = HID // HEADS
BT = 1024


def _leaky(x):
    return jnp.where(x >= 0, x, 0.2 * x)


def _elu(x):
    return jnp.where(x > 0, x, jnp.exp(x) - 1.0)


def _gnn_kernel(t_ref, a_ref, v_ref,
                gl0_ref, gl1_ref, gl2_ref, gr0_ref, gr1_ref, gr2_ref,
                cl_ref, cr_ref, ma1_ref, bias1_ref,
                wl2_ref, bl2_ref, wr2_ref, br2_ref, ma2_ref, bias2_ref,
                out_ref):
    f32 = jnp.float32
    feats = (t_ref[...], a_ref[...], v_ref[...])
    gl = (gl0_ref[...], gl1_ref[...], gl2_ref[...])
    gr = (gr0_ref[...], gr1_ref[...], gr2_ref[...])
    cl = cl_ref[...]
    cr = cr_ref[...]

    xl = [jnp.dot(feats[i], gl[i], preferred_element_type=f32) + cl[i:i + 1]
          for i in range(3)]
    xr = [jnp.dot(feats[i], gr[i], preferred_element_type=f32) + cr[i:i + 1]
          for i in range(3)]

    ma1 = ma1_ref[...]
    bias1 = bias1_ref[...]

    def gat(xli, xri, ma, bias):
        outs = []
        for d in range(3):
            a, b = [s for s in range(3) if s != d]
            ea = _leaky(xli[a] + xri[d])
            eb = _leaky(xli[b] + xri[d])
            dlog = jnp.dot(eb - ea, ma, preferred_element_type=f32)
            sa = 1.0 / (1.0 + jnp.exp(dlog))
            agg = xli[b] + sa * (xli[a] - xli[b])
            outs.append(agg + bias)
        return outs

    h = [_elu(o) for o in gat(xl, xr, ma1, bias1)]

    wl2 = wl2_ref[...]
    wr2 = wr2_ref[...]
    bl2 = bl2_ref[...]
    br2 = br2_ref[...]
    yl = [jnp.dot(h[i], wl2, preferred_element_type=f32) + bl2 for i in range(3)]
    yr = [jnp.dot(h[i], wr2, preferred_element_type=f32) + br2 for i in range(3)]
    o2 = gat(yl, yr, ma2_ref[...], bias2_ref[...])
    out_ref[...] = (o2[0] + o2[1] + o2[2]) * (1.0 / 3.0)


@jax.jit
def kernel(text_features, audio_features, video_features, W_text, b_text,
           W_audio, b_audio, W_video, b_video, Wl1, bl1, Wr1, br1, att1,
           bias1, Wl2, bl2, Wr2, br2, att2, bias2):
    f32 = jnp.float32
    row = lambda v: v.reshape(1, -1).astype(f32)

    Ws = (W_text, W_audio, W_video)
    bs = (b_text, b_audio, b_video)
    gls = [(W.T @ Wl1.T).astype(f32) for W in Ws]
    grs = [(W.T @ Wr1.T).astype(f32) for W in Ws]
    cl = jnp.stack([b @ Wl1.T + bl1 for b in bs]).astype(f32)
    cr = jnp.stack([b @ Wr1.T + br1 for b in bs]).astype(f32)

    att1_flat = att1.reshape(HEADS * C1)
    head = jnp.arange(HID, dtype=jnp.int32) // C1
    same = (head[:, None] == head[None, :]).astype(f32)
    ma1 = (att1_flat[:, None] * same).astype(f32)
    ma2 = jnp.broadcast_to(att2.reshape(HID, 1), (HID, HID)).astype(f32)

    grid = (B // BT,)
    data_spec = pl.BlockSpec((BT, UNI), lambda i: (i, 0))
    w_uni = pl.BlockSpec((UNI, HID), lambda i: (0, 0))
    w_hid = pl.BlockSpec((HID, HID), lambda i: (0, 0))
    c3 = pl.BlockSpec((3, HID), lambda i: (0, 0))
    vec = pl.BlockSpec((1, HID), lambda i: (0, 0))

    out = pl.pallas_call(
        _gnn_kernel,
        grid=grid,
        in_specs=[
            data_spec, data_spec, data_spec,
            w_uni, w_uni, w_uni, w_uni, w_uni, w_uni,
            c3, c3, w_hid, vec,
            w_hid, vec, w_hid, vec, w_hid, vec,
        ],
        out_specs=pl.BlockSpec((BT, HID), lambda i: (i, 0)),
        out_shape=jax.ShapeDtypeStruct((B, HID), f32),
    )(
        text_features, audio_features, video_features,
        gls[0], gls[1], gls[2], grs[0], grs[1], grs[2],
        cl, cr, ma1, row(bias1),
        Wl2.T.astype(f32), row(bl2), Wr2.T.astype(f32), row(br2),
        ma2, row(bias2),
    )
    return out
